# Initial kernel scaffold; baseline (speedup 1.0000x reference)
import jax
import jax.numpy as jnp
from jax.experimental import pallas as pl


def kernel(hidden_states, emb_table, Wq, bq, W1, b1, W2, b2):
    raise NotImplementedError("write your pallas kernel here")


# speedup vs baseline: 2.0539x; 2.0539x over previous
import jax
import jax.numpy as jnp
import numpy as np
from jax.experimental import pallas as pl
from jax.experimental.pallas import tpu as pltpu

_N = 512
_TOPK = 8
_B, _S, _H, _K = 2, 128, 768, 128
_T = _B * _S
_TT = 8
_M = _TT * _N

_BF = jnp.bfloat16
_F32 = jnp.float32
_SQRT_HALF = np.float32(np.sqrt(0.5))

_ERFC_P = [2.326819970068386e-2, -1.387039388740657e-1, 3.687424674597105e-1,
           -5.824733027278666e-1, 6.210004621745983e-1, -4.944515323274145e-1,
           3.404879937665872e-1, -2.741127028184656e-1, 5.638259427386472e-1]
_ERFC_R = [-1.047766399936249e+1, 1.297719955372516e+1, -7.495518717768503e+0,
           2.921019019210786e+0, -1.015265279202700e+0, 4.218463358204948e-1,
           -2.820767439740514e-1, 5.641895067754075e-1]


def _polevl(x, coeffs):
    p = jnp.full_like(x, np.float32(coeffs[0]))
    for c in coeffs[1:]:
        p = p * x + np.float32(c)
    return p


def _gelu_exact(h):
    x = -h * _SQRT_HALF
    ax = jnp.abs(x)
    small = np.float32(1.0) - jax.lax.erf(x)
    z = jnp.exp(-x * x)
    r = np.float32(1.0) / ax
    y2 = r * r
    p = jnp.where(ax < np.float32(2.0), _polevl(y2, _ERFC_P),
                  _polevl(y2, _ERFC_R))
    yv = z * r * p
    yc = jnp.where(-x * x < np.float32(-88.72283905206835),
                   jnp.zeros_like(yv), yv)
    big = jnp.where(x < np.float32(0.0), np.float32(2.0) - yc, yc)
    er = jnp.where(ax > np.float32(1.0), big, small)
    return 0.5 * h * er


def _score_body(q_ref, emb_ref, W1_ref, b1_ref, W2_ref, s_ref):
    qb = q_ref[...].astype(_BF)
    eb = emb_ref[...].astype(_BF)
    qpart = jnp.broadcast_to(qb[:, None, :], (_TT, _N, _K)).reshape(_M, _K)
    epart = jnp.broadcast_to(eb[None, :, :], (_TT, _N, _K)).reshape(_M, _K)
    comb = jnp.concatenate([qpart, epart], axis=1)
    h = jnp.dot(comb, W1_ref[...].astype(_BF),
                preferred_element_type=_F32) + b1_ref[...]
    g = _gelu_exact(h)
    s_ref[...] = jnp.dot(g.astype(_BF), W2_ref[...].astype(_BF),
                         preferred_element_type=_F32)


def _topk_body(s_ref, emb_ref, ts_ref, ke_ref):
    emb = emb_ref[...]
    iota_n = jax.lax.broadcasted_iota(jnp.int32, (_T, _N), 1)
    s = s_ref[...]
    ts_cols, ke_parts = [], []
    for _ in range(_TOPK):
        m = jnp.max(s, axis=1, keepdims=True)
        idx = jnp.min(jnp.where(s >= m, iota_n, _N), axis=1,
                      keepdims=True)
        onehot = (iota_n == idx).astype(_F32)
        ke_parts.append(jnp.dot(onehot, emb,
                                precision=jax.lax.Precision.HIGHEST))
        ts_cols.append(m)
        s = jnp.where(iota_n == idx, -jnp.inf, s)
    ts_ref[...] = jnp.concatenate(ts_cols, axis=1)
    ke_ref[...] = jnp.stack(ke_parts, axis=1)


def kernel(hidden_states, emb_table, Wq, bq, W1, b1, W2, b2):
    queries = hidden_states @ Wq + bq
    q2 = queries.reshape(_T, _K)

    scores_col = pl.pallas_call(
        _score_body,
        grid=(_T // _TT,),
        in_specs=[
            pl.BlockSpec((_TT, _K), lambda i: (i, 0)),
            pl.BlockSpec((_N, _K), lambda i: (0, 0)),
            pl.BlockSpec((2 * _K, _K), lambda i: (0, 0)),
            pl.BlockSpec((1, _K), lambda i: (0, 0)),
            pl.BlockSpec((_K, 1), lambda i: (0, 0)),
        ],
        out_specs=pl.BlockSpec((_M, 1), lambda i: (i, 0)),
        out_shape=jax.ShapeDtypeStruct((_T * _N, 1), _F32),
        compiler_params=pltpu.CompilerParams(
            vmem_limit_bytes=100 * 1024 * 1024),
    )(q2, emb_table, W1, b1.reshape(1, _K), W2)

    s2 = scores_col.reshape(_T, _N)

    ts, ke = pl.pallas_call(
        _topk_body,
        grid=(1,),
        in_specs=[
            pl.BlockSpec((_T, _N), lambda i: (0, 0)),
            pl.BlockSpec((_N, _K), lambda i: (0, 0)),
        ],
        out_specs=[
            pl.BlockSpec((_T, _TOPK), lambda i: (0, 0)),
            pl.BlockSpec((_T, _TOPK, _K), lambda i: (0, 0, 0)),
        ],
        out_shape=[
            jax.ShapeDtypeStruct((_T, _TOPK), _F32),
            jax.ShapeDtypeStruct((_T, _TOPK, _K), _F32),
        ],
        compiler_params=pltpu.CompilerParams(
            vmem_limit_bytes=100 * 1024 * 1024),
    )(s2, emb_table)

    knowledge_embeddings = ke.reshape(_B, _S, _TOPK, _K)
    knowledge_scores = ts.reshape(_B, _S, _TOPK) + b2[0]
    knowledge_mask = jnp.ones((_B, _S, _TOPK), _F32)
    return knowledge_embeddings, knowledge_mask, knowledge_scores
